# Initial kernel scaffold; baseline (speedup 1.0000x reference)
import jax
import jax.numpy as jnp
from jax.experimental import pallas as pl


def kernel(q_in, kv_in, Wq, bq, Wk, bk, Wv, bv, Woff, boff, Wattn, battn, Wout, bout):
    raise NotImplementedError("write your pallas kernel here")


# speedup vs baseline: 14.9754x; 14.9754x over previous
import functools
import math

import jax
import jax.numpy as jnp
from jax import lax
from jax.experimental import pallas as pl
from jax.experimental.pallas import tpu as pltpu
from jax.experimental.pallas import tpu_sc as plsc

H = 16
P = 4


def _proj_body(x_ref, kv_ref, wq_ref, bq_ref, wk_ref, bk_ref, wv_ref, bv_ref,
               woff_ref, boff_ref, wattn_ref, battn_ref,
               qh_ref, k_ref, v_ref, off_ref, lg_ref):
    hd = qh_ref.shape[-1]
    x = x_ref[0]
    kv = kv_ref[0]
    q = jnp.dot(x, wq_ref[...], preferred_element_type=jnp.float32) + bq_ref[...]
    k = jnp.dot(kv, wk_ref[...], preferred_element_type=jnp.float32) + bk_ref[...]
    v = jnp.dot(kv, wv_ref[...], preferred_element_type=jnp.float32) + bv_ref[...]
    off = jnp.dot(x, woff_ref[...], preferred_element_type=jnp.float32) + boff_ref[...]
    lg = jnp.dot(x, wattn_ref[...], preferred_element_type=jnp.float32) + battn_ref[...]
    k_ref[0] = k
    v_ref[0] = v
    off_ref[0] = off
    lg_ref[0] = lg
    scale = 1.0 / math.sqrt(hd)
    for h in range(H):
        qh_ref[0, h] = q[:, h * hd:(h + 1) * hd] * scale


def _projections(q_in, kv_in, Wq, bq, Wk, bk, Wv, bv, Woff, boff, Wattn, battn):
    B, L, D = q_in.shape
    HD = D // H
    LB = 256
    grid = (B, L // LB)
    full = lambda shape: pl.BlockSpec(shape, lambda b, i: (0,) * len(shape))
    row_spec = pl.BlockSpec((1, LB, D), lambda b, i: (b, i, 0))
    out_shapes = (
        jax.ShapeDtypeStruct((B, H, L, HD), jnp.float32),
        jax.ShapeDtypeStruct((B, L, D), jnp.float32),
        jax.ShapeDtypeStruct((B, L, D), jnp.float32),
        jax.ShapeDtypeStruct((B, L, H * P), jnp.float32),
        jax.ShapeDtypeStruct((B, L, H * P), jnp.float32),
    )
    return pl.pallas_call(
        _proj_body,
        grid=grid,
        in_specs=[
            row_spec, row_spec,
            full((D, D)), full((D,)), full((D, D)), full((D,)),
            full((D, D)), full((D,)), full((D, H * P)), full((H * P,)),
            full((D, H * P)), full((H * P,)),
        ],
        out_specs=(
            pl.BlockSpec((1, H, LB, HD), lambda b, i: (b, 0, i, 0)),
            row_spec,
            row_spec,
            pl.BlockSpec((1, LB, H * P), lambda b, i: (b, i, 0)),
            pl.BlockSpec((1, LB, H * P), lambda b, i: (b, i, 0)),
        ),
        out_shape=out_shapes,
    )(q_in, kv_in, Wq, bq, Wk, bk, Wv, bv, Woff, boff, Wattn, battn)


def _build_body(k_ref, kn_ref, v_ref, vn_ref, kvp_ref):
    hd = kvp_ref.shape[-1] // 4
    k = k_ref[0]
    kn = kn_ref[0]
    v = v_ref[0]
    vn = vn_ref[0]
    for h in range(H):
        sl = slice(h * hd, (h + 1) * hd)
        kh = k[:, sl]
        vh = v[:, sl]
        kh1 = jnp.concatenate([kh[1:], kn[:1, sl]], axis=0)
        vh1 = jnp.concatenate([vh[1:], vn[:1, sl]], axis=0)
        kvp_ref[0, h] = jnp.concatenate([kh, kh1, vh, vh1], axis=1)


def _build_pairs(k, v):
    B, L, D = k.shape
    HD = D // H
    LB = 256
    nblk = L // LB
    grid = (B, nblk)
    cur = pl.BlockSpec((1, LB, D), lambda b, i: (b, i, 0))
    nxt = pl.BlockSpec((1, LB, D),
                       lambda b, i: (b, jnp.minimum(i + 1, nblk - 1), 0))
    return pl.pallas_call(
        _build_body,
        grid=grid,
        in_specs=[cur, nxt, cur, nxt],
        out_specs=pl.BlockSpec((1, H, LB, 4 * HD), lambda b, i: (b, 0, i, 0)),
        out_shape=jax.ShapeDtypeStruct((B, H, L, 4 * HD), jnp.float32),
    )(k, k, v, v)


def _attn_body(q_ref, g_ref, meta_ref, ctx_ref):
    hd = q_ref.shape[-1]
    q = q_ref[0, 0]
    g = g_ref[0, 0]
    meta = meta_ref[0, 0]
    scores = []
    for p in range(P):
        o = p * 4 * hd
        k0 = g[:, o:o + hd]
        k1 = g[:, o + hd:o + 2 * hd]
        d0 = jnp.sum(q * k0, axis=1, keepdims=True)
        d1 = jnp.sum(q * k1, axis=1, keepdims=True)
        w0 = meta[:, p:p + 1]
        w1 = meta[:, P + p:P + p + 1]
        lg = meta[:, 2 * P + p:2 * P + p + 1]
        scores.append(d0 * w0 + d1 * w1 + lg)
    m = jnp.maximum(jnp.maximum(scores[0], scores[1]),
                    jnp.maximum(scores[2], scores[3]))
    es = [jnp.exp(s - m) for s in scores]
    z = es[0] + es[1] + es[2] + es[3]
    ctx = jnp.zeros(q.shape, jnp.float32)
    for p in range(P):
        o = p * 4 * hd
        v0 = g[:, o + 2 * hd:o + 3 * hd]
        v1 = g[:, o + 3 * hd:o + 4 * hd]
        w0 = meta[:, p:p + 1]
        w1 = meta[:, P + p:P + p + 1]
        ctx = ctx + (es[p] / z) * (w0 * v0 + w1 * v1)
    ctx_ref[0, 0] = ctx


def _attention(qh, gath, meta):
    B, Hh, L, HD = qh.shape
    LB = 512
    grid = (B, Hh, L // LB)
    return pl.pallas_call(
        _attn_body,
        grid=grid,
        in_specs=[
            pl.BlockSpec((1, 1, LB, HD), lambda b, h, i: (b, h, i, 0)),
            pl.BlockSpec((1, 1, LB, P * 4 * HD), lambda b, h, i: (b, h, i, 0)),
            pl.BlockSpec((1, 1, LB, 64), lambda b, h, i: (b, h, i, 0)),
        ],
        out_specs=pl.BlockSpec((1, 1, LB, HD), lambda b, h, i: (b, h, i, 0)),
        out_shape=jax.ShapeDtypeStruct((B, Hh, L, HD), jnp.float32),
    )(qh, gath, meta)


def _outproj_body(ctx_ref, w_ref, b_ref, o_ref):
    hd = ctx_ref.shape[-1]
    x = jnp.concatenate([ctx_ref[0, h] for h in range(H)], axis=1)
    o_ref[0] = jnp.dot(x, w_ref[...], preferred_element_type=jnp.float32) + b_ref[...]


def _outproj(ctx, Wout, bout):
    B, Hh, L, HD = ctx.shape
    D = Hh * HD
    LB = 256
    grid = (B, L // LB)
    return pl.pallas_call(
        _outproj_body,
        grid=grid,
        in_specs=[
            pl.BlockSpec((1, H, LB, HD), lambda b, i: (b, 0, i, 0)),
            pl.BlockSpec((D, D), lambda b, i: (0, 0)),
            pl.BlockSpec((D,), lambda b, i: (0,)),
        ],
        out_specs=pl.BlockSpec((1, LB, D), lambda b, i: (b, i, 0)),
        out_shape=jax.ShapeDtypeStruct((B, L, D), jnp.float32),
    )(ctx, Wout, bout)


def _sc_gather(table, gidx):
    R = gidx.shape[0]
    W = table.shape[1]
    NC = 2
    NS = 16
    NW = NC * NS
    r_per_w = R // NW
    CH = 128
    n_chunks = r_per_w // CH
    mesh = plsc.VectorSubcoreMesh(core_axis_name="c", subcore_axis_name="s")

    @functools.partial(
        pl.kernel,
        out_type=jax.ShapeDtypeStruct((R, W), jnp.float32),
        mesh=mesh,
        scratch_types=[
            pltpu.VMEM((2, CH), jnp.int32),
            pltpu.VMEM((2, CH, W), jnp.float32),
            pltpu.SemaphoreType.DMA,
        ],
    )
    def gather_kernel(tab_hbm, idx_hbm, out_hbm, idx_v, rows_v, sem):
        wid = lax.axis_index("s") * NC + lax.axis_index("c")
        base = wid * r_per_w

        @pl.loop(0, n_chunks)
        def _(c):
            start = base + c * CH
            slot = lax.rem(c, 2)
            pltpu.sync_copy(idx_hbm.at[pl.ds(start, CH)], idx_v.at[slot])
            pltpu.async_copy(tab_hbm.at[idx_v.at[slot]], rows_v.at[slot],
                             sem).wait()
            pltpu.sync_copy(rows_v.at[slot], out_hbm.at[pl.ds(start, CH)])

    return gather_kernel(table, gidx)


@jax.jit
def kernel(q_in, kv_in, Wq, bq, Wk, bk, Wv, bv, Woff, boff, Wattn, battn,
           Wout, bout):
    B, L, D = q_in.shape
    HD = D // H

    qh, k, v, off, lg = _projections(q_in, kv_in, Wq, bq, Wk, bk, Wv, bv,
                                     Woff, boff, Wattn, battn)
    kvp = _build_pairs(k, v)

    offT = off.reshape(B, L, H, P).transpose(0, 2, 1, 3)
    lgT = lg.reshape(B, L, H, P).transpose(0, 2, 1, 3)
    basef = jnp.arange(L, dtype=jnp.float32).reshape(1, 1, L, 1)
    idxf = jnp.clip(basef + offT, 0.0, float(L - 1))
    base = jnp.clip(jnp.floor(idxf), 0.0, float(L - 2))
    w1 = idxf - base
    w0 = 1.0 - w1
    meta = jnp.concatenate(
        [w0, w1, lgT, jnp.zeros((B, H, L, 64 - 3 * P), jnp.float32)], axis=-1)
    bh = jnp.arange(B * H, dtype=jnp.int32).reshape(B, H, 1, 1)
    rowid = (bh * L + base.astype(jnp.int32)).reshape(-1)

    gath = _sc_gather(kvp.reshape(B * H * L, 4 * HD), rowid)
    gath = gath.reshape(B, H, L, P * 4 * HD)

    ctx = _attention(qh, gath, meta)
    return _outproj(ctx, Wout, bout)


# speedup vs baseline: 27.1067x; 1.8101x over previous
import functools
import math

import jax
import jax.numpy as jnp
from jax import lax
from jax.experimental import pallas as pl
from jax.experimental.pallas import tpu as pltpu
from jax.experimental.pallas import tpu_sc as plsc

H = 16
P = 4


def _proj_body(x_ref, kv_ref, wq_ref, bq_ref, wk_ref, bk_ref, wv_ref, bv_ref,
               woff_ref, boff_ref, wattn_ref, battn_ref,
               qh_ref, k_ref, v_ref, off_ref, lg_ref):
    hd = qh_ref.shape[-1]
    x = x_ref[0]
    kv = kv_ref[0]
    q = jnp.dot(x, wq_ref[...], preferred_element_type=jnp.float32) + bq_ref[...]
    k = jnp.dot(kv, wk_ref[...], preferred_element_type=jnp.float32) + bk_ref[...]
    v = jnp.dot(kv, wv_ref[...], preferred_element_type=jnp.float32) + bv_ref[...]
    off = jnp.dot(x, woff_ref[...], preferred_element_type=jnp.float32) + boff_ref[...]
    lg = jnp.dot(x, wattn_ref[...], preferred_element_type=jnp.float32) + battn_ref[...]
    k_ref[0] = k
    v_ref[0] = v
    off_ref[0] = off
    lg_ref[0] = lg
    scale = 1.0 / math.sqrt(hd)
    for h in range(H):
        qh_ref[0, h] = q[:, h * hd:(h + 1) * hd] * scale


def _projections(q_in, kv_in, Wq, bq, Wk, bk, Wv, bv, Woff, boff, Wattn, battn):
    B, L, D = q_in.shape
    HD = D // H
    LB = 256
    grid = (B, L // LB)
    full = lambda shape: pl.BlockSpec(shape, lambda b, i: (0,) * len(shape))
    row_spec = pl.BlockSpec((1, LB, D), lambda b, i: (b, i, 0))
    out_shapes = (
        jax.ShapeDtypeStruct((B, H, L, HD), jnp.float32),
        jax.ShapeDtypeStruct((B, L, D), jnp.float32),
        jax.ShapeDtypeStruct((B, L, D), jnp.float32),
        jax.ShapeDtypeStruct((B, L, H * P), jnp.float32),
        jax.ShapeDtypeStruct((B, L, H * P), jnp.float32),
    )
    return pl.pallas_call(
        _proj_body,
        grid=grid,
        in_specs=[
            row_spec, row_spec,
            full((D, D)), full((D,)), full((D, D)), full((D,)),
            full((D, D)), full((D,)), full((D, H * P)), full((H * P,)),
            full((D, H * P)), full((H * P,)),
        ],
        out_specs=(
            pl.BlockSpec((1, H, LB, HD), lambda b, i: (b, 0, i, 0)),
            row_spec,
            row_spec,
            pl.BlockSpec((1, LB, H * P), lambda b, i: (b, i, 0)),
            pl.BlockSpec((1, LB, H * P), lambda b, i: (b, i, 0)),
        ),
        out_shape=out_shapes,
    )(q_in, kv_in, Wq, bq, Wk, bk, Wv, bv, Woff, boff, Wattn, battn)


def _build_body(k_ref, kn_ref, v_ref, vn_ref, kvp_ref):
    hd = kvp_ref.shape[-1] // 4
    k = k_ref[0]
    kn = kn_ref[0]
    v = v_ref[0]
    vn = vn_ref[0]
    for h in range(H):
        sl = slice(h * hd, (h + 1) * hd)
        kh = k[:, sl]
        vh = v[:, sl]
        kh1 = jnp.concatenate([kh[1:], kn[:1, sl]], axis=0)
        vh1 = jnp.concatenate([vh[1:], vn[:1, sl]], axis=0)
        kvp_ref[0, h] = jnp.concatenate([kh, kh1, vh, vh1], axis=1)


def _build_pairs(k, v):
    B, L, D = k.shape
    HD = D // H
    LB = 256
    nblk = L // LB
    grid = (B, nblk)
    cur = pl.BlockSpec((1, LB, D), lambda b, i: (b, i, 0))
    nxt = pl.BlockSpec((1, LB, D),
                       lambda b, i: (b, jnp.minimum(i + 1, nblk - 1), 0))
    return pl.pallas_call(
        _build_body,
        grid=grid,
        in_specs=[cur, nxt, cur, nxt],
        out_specs=pl.BlockSpec((1, H, LB, 4 * HD), lambda b, i: (b, 0, i, 0)),
        out_shape=jax.ShapeDtypeStruct((B, H, L, 4 * HD), jnp.float32),
    )(k, k, v, v)


def _attn_body(q_ref, g_ref, meta_ref, ctx_ref):
    hd = q_ref.shape[-1]
    lb = q_ref.shape[-2]
    q = q_ref[0, 0]
    meta = meta_ref[0, 0]
    parts = []
    for p in range(P):
        gp = g_ref[p, 0, 0]
        w0 = meta[:, p:p + 1]
        w1 = meta[:, P + p:P + p + 1]
        parts.append(gp[:, :hd] * w0)
        parts.append(gp[:, hd:2 * hd] * w1)
    K = jnp.concatenate(parts, axis=1)
    QQ = jnp.concatenate([q] * (2 * P), axis=1)
    prod = K * QQ
    seg = jax.lax.broadcasted_iota(jnp.int32, (2 * P * hd, P), 0) // (2 * hd)
    col = jax.lax.broadcasted_iota(jnp.int32, (2 * P * hd, P), 1)
    S = (seg == col).astype(jnp.float32)
    scores = jnp.dot(prod, S, preferred_element_type=jnp.float32)
    scores = scores + meta[:, 2 * P:3 * P]
    m = jnp.max(scores, axis=1, keepdims=True)
    e = jnp.exp(scores - m)
    z = jnp.sum(e, axis=1, keepdims=True)
    wgt = e / z
    ctx = jnp.zeros((lb, hd), jnp.float32)
    for p in range(P):
        gp = g_ref[p, 0, 0]
        c0 = wgt[:, p:p + 1] * meta[:, p:p + 1]
        c1 = wgt[:, p:p + 1] * meta[:, P + p:P + p + 1]
        ctx = ctx + c0 * gp[:, 2 * hd:3 * hd] + c1 * gp[:, 3 * hd:4 * hd]
    ctx_ref[0, 0] = ctx


def _attention(qh, gath5, meta):
    B, Hh, L, HD = qh.shape
    LB = 512
    grid = (B, Hh, L // LB)
    return pl.pallas_call(
        _attn_body,
        grid=grid,
        in_specs=[
            pl.BlockSpec((1, 1, LB, HD), lambda b, h, i: (b, h, i, 0)),
            pl.BlockSpec((P, 1, 1, LB, 4 * HD), lambda b, h, i: (0, b, h, i, 0)),
            pl.BlockSpec((1, 1, LB, 64), lambda b, h, i: (b, h, i, 0)),
        ],
        out_specs=pl.BlockSpec((1, 1, LB, HD), lambda b, h, i: (b, h, i, 0)),
        out_shape=jax.ShapeDtypeStruct((B, Hh, L, HD), jnp.float32),
    )(qh, gath5, meta)


def _outproj_body(ctx_ref, w_ref, b_ref, o_ref):
    hd = ctx_ref.shape[-1]
    x = jnp.concatenate([ctx_ref[0, h] for h in range(H)], axis=1)
    o_ref[0] = jnp.dot(x, w_ref[...], preferred_element_type=jnp.float32) + b_ref[...]


def _outproj(ctx, Wout, bout):
    B, Hh, L, HD = ctx.shape
    D = Hh * HD
    LB = 256
    grid = (B, L // LB)
    return pl.pallas_call(
        _outproj_body,
        grid=grid,
        in_specs=[
            pl.BlockSpec((1, H, LB, HD), lambda b, i: (b, 0, i, 0)),
            pl.BlockSpec((D, D), lambda b, i: (0, 0)),
            pl.BlockSpec((D,), lambda b, i: (0,)),
        ],
        out_specs=pl.BlockSpec((1, LB, D), lambda b, i: (b, i, 0)),
        out_shape=jax.ShapeDtypeStruct((B, L, D), jnp.float32),
    )(ctx, Wout, bout)


def _sc_gather(table, gidx):
    R = gidx.shape[0]
    W = table.shape[1]
    NC = 2
    NS = 16
    NW = NC * NS
    r_per_w = R // NW
    CH = 64
    NBUF = 4
    n_chunks = r_per_w // CH
    mesh = plsc.VectorSubcoreMesh(core_axis_name="c", subcore_axis_name="s")

    @functools.partial(
        pl.kernel,
        out_type=jax.ShapeDtypeStruct((R, W), jnp.float32),
        mesh=mesh,
        scratch_types=[
            pltpu.VMEM((NBUF, CH), jnp.int32),
            pltpu.VMEM((NBUF, CH, W), jnp.float32),
            pltpu.SemaphoreType.DMA((NBUF,)),
            pltpu.SemaphoreType.DMA((NBUF,)),
        ],
    )
    def gather_kernel(tab_hbm, idx_hbm, out_hbm, idx_v, rows_v, sem_g, sem_o):
        wid = lax.axis_index("s") * NC + lax.axis_index("c")
        base = wid * r_per_w

        def fill(c, b):
            pltpu.sync_copy(idx_hbm.at[pl.ds(base + c * CH, CH)], idx_v.at[b])
            pltpu.async_copy(tab_hbm.at[idx_v.at[b]], rows_v.at[b],
                             sem_g.at[b])

        def wait_fill(b):
            pltpu.make_async_copy(tab_hbm.at[idx_v.at[b]], rows_v.at[b],
                                  sem_g.at[b]).wait()

        def drain(c, b):
            pltpu.async_copy(rows_v.at[b], out_hbm.at[pl.ds(base + c * CH, CH)],
                             sem_o.at[b])

        def wait_drain(c, b):
            pltpu.make_async_copy(rows_v.at[b],
                                  out_hbm.at[pl.ds(base + c * CH, CH)],
                                  sem_o.at[b]).wait()

        for b in range(NBUF):
            fill(b, b)

        @pl.loop(0, n_chunks - NBUF, step=NBUF)
        def _(c):
            for b in range(NBUF):
                wait_fill(b)
                drain(c + b, b)
            for b in range(NBUF):
                wait_drain(c + b, b)
                fill(c + NBUF + b, b)

        for b in range(NBUF):
            wait_fill(b)
            drain(n_chunks - NBUF + b, b)
        for b in range(NBUF):
            wait_drain(n_chunks - NBUF + b, b)

    return gather_kernel(table, gidx)


@jax.jit
def kernel(q_in, kv_in, Wq, bq, Wk, bk, Wv, bv, Woff, boff, Wattn, battn,
           Wout, bout):
    B, L, D = q_in.shape
    HD = D // H

    qh, k, v, off, lg = _projections(q_in, kv_in, Wq, bq, Wk, bk, Wv, bv,
                                     Woff, boff, Wattn, battn)
    kvp = _build_pairs(k, v)

    offT = off.reshape(B, L, H, P).transpose(0, 2, 1, 3)
    lgT = lg.reshape(B, L, H, P).transpose(0, 2, 1, 3)
    basef = jnp.arange(L, dtype=jnp.float32).reshape(1, 1, L, 1)
    idxf = jnp.clip(basef + offT, 0.0, float(L - 1))
    base = jnp.clip(jnp.floor(idxf), 0.0, float(L - 2))
    w1 = idxf - base
    w0 = 1.0 - w1
    meta = jnp.concatenate(
        [w0, w1, lgT, jnp.zeros((B, H, L, 64 - 3 * P), jnp.float32)], axis=-1)
    bh = jnp.arange(B * H, dtype=jnp.int32).reshape(B, H, 1, 1)
    rowid = (bh * L + base.astype(jnp.int32)).transpose(3, 0, 1, 2).reshape(-1)

    gath = _sc_gather(kvp.reshape(B * H * L, 4 * HD), rowid)
    gath5 = gath.reshape(P, B, H, L, 4 * HD)

    ctx = _attention(qh, gath5, meta)
    return _outproj(ctx, Wout, bout)
